# baseline (speedup 1.0000x reference)
import functools

import jax
import jax.numpy as jnp
from jax import lax
from jax.experimental import pallas as pl
from jax.experimental.pallas import tpu as pltpu
from jax.experimental.pallas import tpu_sc as plsc

NC = 2
NS = 16
NW = NC * NS
CH = 128
CR = 2
CB = CR * CH
NSLOT = 4


def _matmul_body(f_ref, w_ref, o_ref, *, deg_col):
    q = jnp.dot(f_ref[...], w_ref[...], preferred_element_type=jnp.float32)
    col = lax.broadcasted_iota(jnp.int32, q.shape, 1)
    o_ref[...] = q + (col == deg_col).astype(jnp.float32)


def _combine_body(p0_ref, p1_ref, q_ref, b_ref, o_ref, *, n_cls):
    pp = p0_ref[:, :n_cls + 1] + p1_ref[:, :n_cls + 1]
    deg = jnp.maximum(pp[:, n_cls:n_cls + 1], 1.0)
    res = pp[:, :n_cls] / deg + q_ref[:, :n_cls] + b_ref[...]
    o_ref[...] = res.T


def _last_for_slot(nk, s):
    return ((nk - 1 - s) // NSLOT) * NSLOT + s


def _make_sc_segsum(n, cpad, e):
    total_blocks = e // CB
    stage = -(-total_blocks // NW)
    rpt = n // NS
    mesh = plsc.VectorSubcoreMesh(core_axis_name="c", subcore_axis_name="s")

    @functools.partial(
        pl.kernel,
        mesh=mesh,
        compiler_params=pltpu.CompilerParams(use_tc_tiling_on_sc=False),
        out_type=jax.ShapeDtypeStruct((NC * n, 128), jnp.float32),
        scratch_types=(
            [pltpu.VMEM((stage, CB), jnp.int32)] * 2
            + [pltpu.VMEM((CB, cpad), jnp.float32) for _ in range(NSLOT)]
            + [pltpu.VMEM_SHARED((n, cpad), jnp.float32)]
            + [pltpu.SemaphoreType.DMA for _ in range(2 * NSLOT)]
        ),
    )
    def segsum(q_hbm, g_hbm, out_hbm, src_v, dst_v, *rest):
        cid = lax.axis_index("c")
        sid = lax.axis_index("s")
        wid = cid * NS + sid
        rows_r = rest[0:NSLOT]
        acc = rest[NSLOT]
        sems = rest[NSLOT + 1:]
        gsem = sems[0:NSLOT]
        ssem = sems[NSLOT:2 * NSLOT]

        b0 = wid * total_blocks // NW
        nk = (wid + 1) * total_blocks // NW - b0

        zrow = jnp.zeros((16,), jnp.float32)

        def zloop(i, carry):
            for c in range(cpad // 16):
                rows_r[0][i, pl.ds(16 * c, 16)] = zrow
            return carry

        lax.fori_loop(0, CB, zloop, 0)
        nfull, tailr = rpt // CB, rpt % CB
        for z in range(nfull):
            pltpu.sync_copy(rows_r[0],
                            acc.at[pl.ds(sid * rpt + z * CB, CB)])
        if tailr:
            pltpu.sync_copy(rows_r[0].at[pl.ds(0, tailr)],
                            acc.at[pl.ds(sid * rpt + nfull * CB, tailr)])

        pltpu.sync_copy(g_hbm.at[0, pl.ds(b0, stage)], src_v)
        pltpu.sync_copy(g_hbm.at[1, pl.ds(b0, stage)], dst_v)

        def issue_gather(k, s):
            pltpu.async_copy(q_hbm.at[src_v.at[k]], rows_r[s], gsem[s])

        def wait_gather(k, s):
            pltpu.make_async_copy(q_hbm.at[src_v.at[k]], rows_r[s],
                                  gsem[s]).wait()

        def issue_scatter(k, s):
            pltpu.async_copy(rows_r[s], acc.at[dst_v.at[k]], ssem[s],
                             add=True)

        def wait_scatter(k, s):
            pltpu.make_async_copy(rows_r[s], acc.at[dst_v.at[k]],
                                  ssem[s]).wait()

        for s in range(NSLOT):
            @pl.when(s < nk)
            def _(s=s):
                issue_gather(s, s)

        plsc.subcore_barrier()

        def body(j, carry):
            for s in range(NSLOT):
                i = NSLOT * j + s
                wait_gather(i, s)
                issue_scatter(i, s)
            for s in range(NSLOT):
                nxt = NSLOT * j + NSLOT + s

                @pl.when(nxt < nk)
                def _(s=s, nxt=nxt):
                    wait_scatter(nxt - NSLOT, s)
                    issue_gather(nxt, s)
            return carry

        lax.fori_loop(0, nk // NSLOT, body, 0)

        tail_base = (nk // NSLOT) * NSLOT
        for s in range(NSLOT):
            @pl.when(tail_base + s < nk)
            def _(s=s, i=tail_base + s):
                wait_gather(i, s)
                issue_scatter(i, s)
        for s in range(NSLOT):
            @pl.when(s < nk)
            def _(s=s):
                wait_scatter(_last_for_slot(nk, s), s)

        plsc.subcore_barrier()
        pltpu.sync_copy(acc.at[pl.ds(sid * rpt, rpt)],
                        out_hbm.at[pl.ds(cid * n + sid * rpt, rpt),
                                   pl.ds(0, cpad)])

    return segsum


def kernel(features, g, task, W_pred, b_pred):
    n, d = features.shape
    n_cls = W_pred.shape[1]
    e = g.shape[1]
    del task

    cpad = 48
    bm = 5000

    w_pad = jnp.pad(W_pred, ((0, 0), (0, cpad - n_cls)))
    g3 = g.reshape(2, e // CB, CB)

    q = pl.pallas_call(
        functools.partial(_matmul_body, deg_col=n_cls),
        grid=(n // bm,),
        in_specs=[pl.BlockSpec((bm, d), lambda i: (i, 0)),
                  pl.BlockSpec((d, cpad), lambda i: (0, 0))],
        out_specs=pl.BlockSpec((bm, cpad), lambda i: (i, 0)),
        out_shape=jax.ShapeDtypeStruct((n, cpad), jnp.float32),
    )(features, w_pad)

    partials = _make_sc_segsum(n, cpad, e)(q, g3)

    logits_t = pl.pallas_call(
        functools.partial(_combine_body, n_cls=n_cls),
        grid=(1,),
        in_specs=[pl.BlockSpec((n, 128), lambda i: (0, 0)),
                  pl.BlockSpec((n, 128), lambda i: (1, 0)),
                  pl.BlockSpec((n, cpad), lambda i: (0, 0)),
                  pl.BlockSpec((1, n_cls), lambda i: (0, 0))],
        out_specs=pl.BlockSpec((n_cls, n), lambda i: (0, 0)),
        out_shape=jax.ShapeDtypeStruct((n_cls, n), jnp.float32),
    )(partials, partials, q, b_pred.reshape(1, n_cls))
    return logits_t.T


# speedup vs baseline: 1.0264x; 1.0264x over previous
import functools

import jax
import jax.numpy as jnp
from jax import lax
from jax.experimental import pallas as pl
from jax.experimental.pallas import tpu as pltpu
from jax.experimental.pallas import tpu_sc as plsc

NC = 2
NS = 16
NW = NC * NS
CH = 128
CR = 1
CB = CR * CH
NSLOT = 8


def _matmul_body(f_ref, w_ref, o_ref, *, deg_col):
    q = jnp.dot(f_ref[...], w_ref[...], preferred_element_type=jnp.float32)
    col = lax.broadcasted_iota(jnp.int32, q.shape, 1)
    o_ref[...] = q + (col == deg_col).astype(jnp.float32)


def _combine_body(p0_ref, p1_ref, q_ref, b_ref, o_ref, *, n_cls):
    pp = p0_ref[:, :n_cls + 1] + p1_ref[:, :n_cls + 1]
    deg = jnp.maximum(pp[:, n_cls:n_cls + 1], 1.0)
    res = pp[:, :n_cls] / deg + q_ref[:, :n_cls] + b_ref[...]
    o_ref[...] = res.T


def _last_for_slot(nk, s):
    return ((nk - 1 - s) // NSLOT) * NSLOT + s


def _make_sc_segsum(n, cpad, e):
    total_blocks = e // CB
    stage = -(-total_blocks // NW)
    rpt = n // NS
    mesh = plsc.VectorSubcoreMesh(core_axis_name="c", subcore_axis_name="s")

    @functools.partial(
        pl.kernel,
        mesh=mesh,
        compiler_params=pltpu.CompilerParams(use_tc_tiling_on_sc=False),
        out_type=jax.ShapeDtypeStruct((NC * n, 128), jnp.float32),
        scratch_types=(
            [pltpu.VMEM((stage, CB), jnp.int32)] * 2
            + [pltpu.VMEM((CB, cpad), jnp.float32) for _ in range(NSLOT)]
            + [pltpu.VMEM_SHARED((n, cpad), jnp.float32)]
            + [pltpu.SemaphoreType.DMA for _ in range(2 * NSLOT)]
        ),
    )
    def segsum(q_hbm, g_hbm, out_hbm, src_v, dst_v, *rest):
        cid = lax.axis_index("c")
        sid = lax.axis_index("s")
        wid = cid * NS + sid
        rows_r = rest[0:NSLOT]
        acc = rest[NSLOT]
        sems = rest[NSLOT + 1:]
        gsem = sems[0:NSLOT]
        ssem = sems[NSLOT:2 * NSLOT]

        b0 = wid * total_blocks // NW
        nk = (wid + 1) * total_blocks // NW - b0

        zrow = jnp.zeros((16,), jnp.float32)

        def zloop(i, carry):
            for c in range(cpad // 16):
                rows_r[0][i, pl.ds(16 * c, 16)] = zrow
            return carry

        lax.fori_loop(0, CB, zloop, 0)
        nfull, tailr = rpt // CB, rpt % CB
        for z in range(nfull):
            pltpu.sync_copy(rows_r[0],
                            acc.at[pl.ds(sid * rpt + z * CB, CB)])
        if tailr:
            pltpu.sync_copy(rows_r[0].at[pl.ds(0, tailr)],
                            acc.at[pl.ds(sid * rpt + nfull * CB, tailr)])

        pltpu.sync_copy(g_hbm.at[0, pl.ds(b0, stage)], src_v)
        pltpu.sync_copy(g_hbm.at[1, pl.ds(b0, stage)], dst_v)

        def issue_gather(k, s):
            pltpu.async_copy(q_hbm.at[src_v.at[k]], rows_r[s], gsem[s])

        def wait_gather(k, s):
            pltpu.make_async_copy(q_hbm.at[src_v.at[k]], rows_r[s],
                                  gsem[s]).wait()

        def issue_scatter(k, s):
            pltpu.async_copy(rows_r[s], acc.at[dst_v.at[k]], ssem[s],
                             add=True)

        def wait_scatter(k, s):
            pltpu.make_async_copy(rows_r[s], acc.at[dst_v.at[k]],
                                  ssem[s]).wait()

        for s in range(NSLOT):
            @pl.when(s < nk)
            def _(s=s):
                issue_gather(s, s)

        plsc.subcore_barrier()

        def body(j, carry):
            for s in range(NSLOT):
                i = NSLOT * j + s
                wait_gather(i, s)
                issue_scatter(i, s)
            for s in range(NSLOT):
                nxt = NSLOT * j + NSLOT + s

                @pl.when(nxt < nk)
                def _(s=s, nxt=nxt):
                    wait_scatter(nxt - NSLOT, s)
                    issue_gather(nxt, s)
            return carry

        lax.fori_loop(0, nk // NSLOT, body, 0)

        tail_base = (nk // NSLOT) * NSLOT
        for s in range(NSLOT):
            @pl.when(tail_base + s < nk)
            def _(s=s, i=tail_base + s):
                wait_gather(i, s)
                issue_scatter(i, s)
        for s in range(NSLOT):
            @pl.when(s < nk)
            def _(s=s):
                wait_scatter(_last_for_slot(nk, s), s)

        plsc.subcore_barrier()
        pltpu.sync_copy(acc.at[pl.ds(sid * rpt, rpt)],
                        out_hbm.at[pl.ds(cid * n + sid * rpt, rpt),
                                   pl.ds(0, cpad)])

    return segsum


def kernel(features, g, task, W_pred, b_pred):
    n, d = features.shape
    n_cls = W_pred.shape[1]
    e = g.shape[1]
    del task

    cpad = 48
    bm = 10000

    w_pad = jnp.pad(W_pred, ((0, 0), (0, cpad - n_cls)))
    g3 = g.reshape(2, e // CB, CB)

    q = pl.pallas_call(
        functools.partial(_matmul_body, deg_col=n_cls),
        grid=(n // bm,),
        in_specs=[pl.BlockSpec((bm, d), lambda i: (i, 0)),
                  pl.BlockSpec((d, cpad), lambda i: (0, 0))],
        out_specs=pl.BlockSpec((bm, cpad), lambda i: (i, 0)),
        out_shape=jax.ShapeDtypeStruct((n, cpad), jnp.float32),
    )(features, w_pad)

    partials = _make_sc_segsum(n, cpad, e)(q, g3)

    logits_t = pl.pallas_call(
        functools.partial(_combine_body, n_cls=n_cls),
        grid=(1,),
        in_specs=[pl.BlockSpec((n, 128), lambda i: (0, 0)),
                  pl.BlockSpec((n, 128), lambda i: (1, 0)),
                  pl.BlockSpec((n, cpad), lambda i: (0, 0)),
                  pl.BlockSpec((1, n_cls), lambda i: (0, 0))],
        out_specs=pl.BlockSpec((n_cls, n), lambda i: (0, 0)),
        out_shape=jax.ShapeDtypeStruct((n_cls, n), jnp.float32),
    )(partials, partials, q, b_pred.reshape(1, n_cls))
    return logits_t.T


# speedup vs baseline: 1.0363x; 1.0097x over previous
import functools

import jax
import jax.numpy as jnp
from jax import lax
from jax.experimental import pallas as pl
from jax.experimental.pallas import tpu as pltpu
from jax.experimental.pallas import tpu_sc as plsc

NC = 2
NS = 16
NW = NC * NS
CH = 128
CR = 1
CB = CR * CH
NSLOT = 8


def _matmul_body(f_ref, w_ref, o_ref, *, deg_col):
    q = jnp.dot(f_ref[...], w_ref[...], preferred_element_type=jnp.float32)
    col = lax.broadcasted_iota(jnp.int32, q.shape, 1)
    o_ref[...] = q + (col == deg_col).astype(jnp.float32)


def _combine_body(p0_ref, p1_ref, q_ref, b_ref, o_ref, *, n_cls):
    pp = p0_ref[:, :n_cls + 1] + p1_ref[:, :n_cls + 1]
    deg = jnp.maximum(pp[:, n_cls:n_cls + 1], 1.0)
    res = pp[:, :n_cls] / deg + q_ref[:, :n_cls] + b_ref[...]
    o_ref[...] = res.T


def _last_for_slot(nk, s):
    return ((nk - 1 - s) // NSLOT) * NSLOT + s


def _make_sc_segsum(n, cpad, e):
    total_blocks = e // CB
    stage = -(-total_blocks // NW)
    rpt = n // NS
    mesh = plsc.VectorSubcoreMesh(core_axis_name="c", subcore_axis_name="s")

    @functools.partial(
        pl.kernel,
        mesh=mesh,
        compiler_params=pltpu.CompilerParams(use_tc_tiling_on_sc=False),
        out_type=jax.ShapeDtypeStruct((NC * n, 128), jnp.float32),
        scratch_types=(
            [pltpu.VMEM((stage, CB), jnp.int32)] * 2
            + [pltpu.VMEM((CB, cpad), jnp.float32) for _ in range(NSLOT)]
            + [pltpu.VMEM_SHARED((n, cpad), jnp.float32)]
            + [pltpu.SemaphoreType.DMA for _ in range(2 * NSLOT)]
        ),
    )
    def segsum(q_hbm, g_hbm, out_hbm, src_v, dst_v, *rest):
        cid = lax.axis_index("c")
        sid = lax.axis_index("s")
        wid = cid * NS + sid
        rows_r = rest[0:NSLOT]
        acc = rest[NSLOT]
        sems = rest[NSLOT + 1:]
        gsem = sems[0:NSLOT]
        ssem = sems[NSLOT:2 * NSLOT]

        b0 = wid * total_blocks // NW
        nk = (wid + 1) * total_blocks // NW - b0

        zrow = jnp.zeros((16,), jnp.float32)

        def zloop(i, carry):
            for c in range(cpad // 16):
                rows_r[0][i, pl.ds(16 * c, 16)] = zrow
            return carry

        lax.fori_loop(0, CB, zloop, 0)
        nfull, tailr = rpt // CB, rpt % CB
        for z in range(nfull):
            pltpu.sync_copy(rows_r[0],
                            acc.at[pl.ds(sid * rpt + z * CB, CB)])
        if tailr:
            pltpu.sync_copy(rows_r[0].at[pl.ds(0, tailr)],
                            acc.at[pl.ds(sid * rpt + nfull * CB, tailr)])

        pltpu.sync_copy(g_hbm.at[0, pl.ds(b0, stage)], src_v)
        pltpu.sync_copy(g_hbm.at[1, pl.ds(b0, stage)], dst_v)

        def issue_gather(k, s):
            pltpu.async_copy(q_hbm.at[src_v.at[k]], rows_r[s], gsem[s])

        def wait_gather(k, s):
            pltpu.make_async_copy(q_hbm.at[src_v.at[k]], rows_r[s],
                                  gsem[s]).wait()

        def issue_scatter(k, s):
            pltpu.async_copy(rows_r[s], acc.at[dst_v.at[k]], ssem[s],
                             add=True)

        def wait_scatter(k, s):
            pltpu.make_async_copy(rows_r[s], acc.at[dst_v.at[k]],
                                  ssem[s]).wait()

        for s in range(NSLOT):
            @pl.when(s < nk)
            def _(s=s):
                issue_gather(s, s)

        plsc.subcore_barrier()

        def body(j, carry):
            for s in range(NSLOT):
                i = NSLOT * j + s
                wait_gather(i, s)
                issue_scatter(i, s)
            for s in range(NSLOT):
                nxt = NSLOT * j + NSLOT + s

                @pl.when(nxt < nk)
                def _(s=s, nxt=nxt):
                    wait_scatter(nxt - NSLOT, s)
                    issue_gather(nxt, s)
            return carry

        lax.fori_loop(0, nk // NSLOT, body, 0)

        tail_base = (nk // NSLOT) * NSLOT
        for s in range(NSLOT):
            @pl.when(tail_base + s < nk)
            def _(s=s, i=tail_base + s):
                wait_gather(i, s)
                issue_scatter(i, s)
        for s in range(NSLOT):
            @pl.when(s < nk)
            def _(s=s):
                wait_scatter(_last_for_slot(nk, s), s)

        plsc.subcore_barrier()
        pltpu.sync_copy(acc.at[pl.ds(sid * rpt, rpt)],
                        out_hbm.at[pl.ds(cid * n + sid * rpt, rpt),
                                   pl.ds(0, cpad)])

    return segsum


def kernel(features, g, task, W_pred, b_pred):
    n, d = features.shape
    n_cls = W_pred.shape[1]
    e = g.shape[1]
    del task

    cpad = 48
    bm = 5000

    w_pad = jnp.pad(W_pred, ((0, 0), (0, cpad - n_cls)))
    g3 = g.reshape(2, e // CB, CB)

    q = pl.pallas_call(
        functools.partial(_matmul_body, deg_col=n_cls),
        grid=(n // bm,),
        in_specs=[pl.BlockSpec((bm, d), lambda i: (i, 0)),
                  pl.BlockSpec((d, cpad), lambda i: (0, 0))],
        out_specs=pl.BlockSpec((bm, cpad), lambda i: (i, 0)),
        out_shape=jax.ShapeDtypeStruct((n, cpad), jnp.float32),
    )(features, w_pad)

    partials = _make_sc_segsum(n, cpad, e)(q, g3)

    logits_t = pl.pallas_call(
        functools.partial(_combine_body, n_cls=n_cls),
        grid=(1,),
        in_specs=[pl.BlockSpec((n, 128), lambda i: (0, 0)),
                  pl.BlockSpec((n, 128), lambda i: (1, 0)),
                  pl.BlockSpec((n, cpad), lambda i: (0, 0)),
                  pl.BlockSpec((1, n_cls), lambda i: (0, 0))],
        out_specs=pl.BlockSpec((n_cls, n), lambda i: (0, 0)),
        out_shape=jax.ShapeDtypeStruct((n_cls, n), jnp.float32),
    )(partials, partials, q, b_pred.reshape(1, n_cls))
    return logits_t.T
